# baseline (speedup 1.0000x reference)
import functools

import numpy as np
import jax
import jax.numpy as jnp
from jax import lax
from jax.experimental import pallas as pl
from jax.experimental.pallas import tpu as pltpu
from jax.experimental.pallas import tpu_sc as plsc

BINS = 32
BATCH = 64
T_TAB = 2048
T_IMG = 4096
M = T_TAB + T_IMG
FEAT = 128
BT = 128
G_TAB = T_TAB // BT
G_IMG = T_IMG // BT
G = G_TAB + G_IMG

NC = 2
NS = 16
NW = NC * NS
LANES = 16
CHUNK = 128
NCHUNK = M // CHUNK
VPC = CHUNK // LANES

_KLOGK = np.zeros(80, np.float32)
_KLOGK[1:BATCH + 1] = (np.arange(1, BATCH + 1, dtype=np.float64)
                       * np.log(np.arange(1, BATCH + 1, dtype=np.float64))
                       ).astype(np.float32)
_LOG_BATCH = np.float32(np.log(float(BATCH)))


def _mean_body(tab_ref, img_ref, x_ref, par_ref, mm_s):
    i = pl.program_id(0)

    def reduce_block(ref):
        s = jnp.sum(ref[...], axis=2) * np.float32(1.0 / FEAT)
        for c in range(BT // CHUNK):
            x_ref[c] = s[:, c * CHUNK:(c + 1) * CHUNK]
        pmin = jnp.min(s)
        pmax = jnp.max(s)

        @pl.when(i == 0)
        def _():
            mm_s[0, 0] = pmin
            mm_s[0, 1] = pmax

        @pl.when(i > 0)
        def _():
            mm_s[0, 0] = jnp.minimum(mm_s[0, 0], pmin)
            mm_s[0, 1] = jnp.maximum(mm_s[0, 1], pmax)

    @pl.when(i < G_TAB)
    def _():
        reduce_block(tab_ref)

    @pl.when(i >= G_TAB)
    def _():
        reduce_block(img_ref)

    @pl.when(i == G - 1)
    def _():
        xmin = mm_s[0, 0]
        scale = np.float32(BINS) / (mm_s[0, 1] - xmin)
        par_ref[...] = jnp.stack([
            jnp.full((LANES,), xmin, jnp.float32),
            jnp.full((LANES,), scale, jnp.float32),
        ])


def _means_params(tab_tokens, img_tokens, interpret=False):
    return pl.pallas_call(
        _mean_body,
        grid=(G,),
        in_specs=[
            pl.BlockSpec((BATCH, BT, FEAT),
                         lambda i: (0, jnp.minimum(i, G_TAB - 1), 0)),
            pl.BlockSpec((BATCH, BT, FEAT),
                         lambda i: (0, jnp.clip(i - G_TAB, 0, G_IMG - 1), 0)),
        ],
        out_specs=[
            pl.BlockSpec((BT // CHUNK, BATCH, CHUNK), lambda i: (i, 0, 0)),
            pl.BlockSpec((2, LANES), lambda i: (0, 0)),
        ],
        out_shape=[
            jax.ShapeDtypeStruct((NCHUNK, BATCH, CHUNK), jnp.float32),
            jax.ShapeDtypeStruct((2, LANES), jnp.float32),
        ],
        scratch_shapes=[
            pltpu.SMEM((1, 2), jnp.float32),
        ],
        interpret=interpret,
    )(tab_tokens, img_tokens)


def _sc_hist_body(x_hbm, params_hbm, tab_hbm, out_hbm,
                  xv0, xv1, pv, tv, cv, ev, sem0, sem1):
    wid = lax.axis_index("s") * NC + lax.axis_index("c")
    two = wid + NW < NCHUNK

    cp0 = pltpu.async_copy(x_hbm.at[wid], xv0, sem0)

    @pl.when(two)
    def _():
        pltpu.async_copy(x_hbm.at[wid + NW], xv1, sem1)

    pltpu.sync_copy(params_hbm, pv)
    pltpu.sync_copy(tab_hbm, tv)

    noff = pv[0, :] * pv[1, :]
    scl = pv[1, :]
    zero16 = jnp.zeros((LANES,), jnp.float32)
    one16 = jnp.full((LANES,), 1.0, jnp.float32)
    lane = lax.iota(jnp.int32, LANES)
    cap = jnp.full((LANES,), BINS - 1, jnp.int32)

    def process_chunk(xv, base):
        def zero_body(j, _):
            cv[pl.ds(pl.multiple_of(j * LANES, LANES), LANES)] = zero16
            return 0

        lax.fori_loop(0, BINS * CHUNK // LANES, zero_body, 0, unroll=8)

        def hist_row(r, _):
            for v in range(VPC):
                x = xv[r, pl.ds(v * LANES, LANES)]
                t = x * scl - noff
                b = jnp.minimum(t.astype(jnp.int32), cap)
                flat = b * CHUNK + (lane + v * LANES)
                plsc.addupdate_scatter(cv, [flat], one16)
            return 0

        lax.fori_loop(0, BATCH, hist_row, 0, unroll=2)

        for v in range(VPC):
            acc0 = zero16
            acc1 = zero16
            for b in range(0, BINS, 2):
                c0 = cv[pl.ds(b * CHUNK + v * LANES, LANES)]
                c1 = cv[pl.ds((b + 1) * CHUNK + v * LANES, LANES)]
                acc0 = acc0 + plsc.load_gather(tv, [c0.astype(jnp.int32)])
                acc1 = acc1 + plsc.load_gather(tv, [c1.astype(jnp.int32)])
            ev[pl.ds(v * LANES, LANES)] = (
                jnp.full((LANES,), _LOG_BATCH, jnp.float32)
                - (acc0 + acc1) * np.float32(1.0 / BATCH))

        pltpu.sync_copy(ev, out_hbm.at[pl.ds(base, CHUNK)])

    cp0.wait()
    process_chunk(xv0, pl.multiple_of(wid * CHUNK, CHUNK))

    @pl.when(two)
    def _():
        pltpu.make_async_copy(x_hbm.at[wid + NW], xv1, sem1).wait()
        process_chunk(xv1, pl.multiple_of((wid + NW) * CHUNK, CHUNK))


def _sc_hist(x, params, tab, interpret=False):
    mesh = plsc.VectorSubcoreMesh(core_axis_name="c", subcore_axis_name="s")
    return pl.kernel(
        _sc_hist_body,
        out_type=jax.ShapeDtypeStruct((M,), jnp.float32),
        mesh=mesh,
        scratch_types=[
            pltpu.VMEM((BATCH, CHUNK), jnp.float32),
            pltpu.VMEM((BATCH, CHUNK), jnp.float32),
            pltpu.VMEM((2, LANES), jnp.float32),
            pltpu.VMEM((80,), jnp.float32),
            pltpu.VMEM((BINS * CHUNK,), jnp.float32),
            pltpu.VMEM((CHUNK,), jnp.float32),
            pltpu.SemaphoreType.DMA,
            pltpu.SemaphoreType.DMA,
        ],
        compiler_params=pltpu.CompilerParams(needs_layout_passes=False),
        interpret=interpret,
    )(x, params, tab)


def _kernel_impl(tab_tokens, img_tokens, interpret=False):
    x, params = _means_params(tab_tokens, img_tokens, interpret=interpret)
    tab = jnp.asarray(_KLOGK)
    return _sc_hist(x, params, tab, interpret=interpret)


def kernel(tab_tokens, img_tokens):
    return _kernel_impl(tab_tokens, img_tokens)


# speedup vs baseline: 1.1020x; 1.1020x over previous
import functools

import numpy as np
import jax
import jax.numpy as jnp
from jax import lax
from jax.experimental import pallas as pl
from jax.experimental.pallas import tpu as pltpu
from jax.experimental.pallas import tpu_sc as plsc

BINS = 32
BATCH = 64
T_TAB = 2048
T_IMG = 4096
M = T_TAB + T_IMG
FEAT = 128
BT = 256
G_TAB = T_TAB // BT
G_IMG = T_IMG // BT
G = G_TAB + G_IMG

NC = 2
NS = 16
NW = NC * NS
LANES = 16
CHUNK = 128
NCHUNK = M // CHUNK
VPC = CHUNK // LANES

_KLOGK = np.zeros(80, np.float32)
_KLOGK[1:BATCH + 1] = (np.arange(1, BATCH + 1, dtype=np.float64)
                       * np.log(np.arange(1, BATCH + 1, dtype=np.float64))
                       ).astype(np.float32)
_LOG_BATCH = np.float32(np.log(float(BATCH)))


def _mean_body(tab_ref, img_ref, x_ref, par_ref, mm_s):
    i = pl.program_id(0)

    def reduce_block(ref):
        s = jnp.sum(ref[...], axis=2) * np.float32(1.0 / FEAT)
        for c in range(BT // CHUNK):
            x_ref[c] = s[:, c * CHUNK:(c + 1) * CHUNK]
        pmin = jnp.min(s)
        pmax = jnp.max(s)

        @pl.when(i == 0)
        def _():
            mm_s[0, 0] = pmin
            mm_s[0, 1] = pmax

        @pl.when(i > 0)
        def _():
            mm_s[0, 0] = jnp.minimum(mm_s[0, 0], pmin)
            mm_s[0, 1] = jnp.maximum(mm_s[0, 1], pmax)

    @pl.when(i < G_TAB)
    def _():
        reduce_block(tab_ref)

    @pl.when(i >= G_TAB)
    def _():
        reduce_block(img_ref)

    @pl.when(i == G - 1)
    def _():
        xmin = mm_s[0, 0]
        scale = np.float32(BINS) / (mm_s[0, 1] - xmin)
        par_ref[...] = jnp.stack([
            jnp.full((LANES,), xmin, jnp.float32),
            jnp.full((LANES,), scale, jnp.float32),
        ])


def _means_params(tab_tokens, img_tokens, interpret=False):
    return pl.pallas_call(
        _mean_body,
        grid=(G,),
        in_specs=[
            pl.BlockSpec((BATCH, BT, FEAT),
                         lambda i: (0, jnp.minimum(i, G_TAB - 1), 0)),
            pl.BlockSpec((BATCH, BT, FEAT),
                         lambda i: (0, jnp.clip(i - G_TAB, 0, G_IMG - 1), 0)),
        ],
        out_specs=[
            pl.BlockSpec((BT // CHUNK, BATCH, CHUNK), lambda i: (i, 0, 0)),
            pl.BlockSpec((2, LANES), lambda i: (0, 0)),
        ],
        out_shape=[
            jax.ShapeDtypeStruct((NCHUNK, BATCH, CHUNK), jnp.float32),
            jax.ShapeDtypeStruct((2, LANES), jnp.float32),
        ],
        scratch_shapes=[
            pltpu.SMEM((1, 2), jnp.float32),
        ],
        interpret=interpret,
    )(tab_tokens, img_tokens)


def _sc_hist_body(x_hbm, params_hbm, tab_hbm, out_hbm,
                  xv0, xv1, pv, tv, cv, ev, sem0, sem1):
    wid = lax.axis_index("s") * NC + lax.axis_index("c")
    two = wid + NW < NCHUNK

    cp0 = pltpu.async_copy(x_hbm.at[wid], xv0, sem0)

    @pl.when(two)
    def _():
        pltpu.async_copy(x_hbm.at[wid + NW], xv1, sem1)

    pltpu.sync_copy(params_hbm, pv)
    pltpu.sync_copy(tab_hbm, tv)

    noff = pv[0, :] * pv[1, :]
    scl = pv[1, :]
    zero16 = jnp.zeros((LANES,), jnp.float32)
    one16 = jnp.full((LANES,), 1.0, jnp.float32)
    lane = lax.iota(jnp.int32, LANES)
    cap = jnp.full((LANES,), BINS - 1, jnp.int32)

    def process_chunk(xv, base):
        def zero_body(j, _):
            cv[pl.ds(pl.multiple_of(j * LANES, LANES), LANES)] = zero16
            return 0

        lax.fori_loop(0, BINS * CHUNK // LANES, zero_body, 0, unroll=8)

        def hist_row(r, _):
            for v in range(VPC):
                x = xv[r, pl.ds(v * LANES, LANES)]
                t = x * scl - noff
                b = jnp.minimum(t.astype(jnp.int32), cap)
                flat = b * CHUNK + (lane + v * LANES)
                plsc.addupdate_scatter(cv, [flat], one16)
            return 0

        lax.fori_loop(0, BATCH, hist_row, 0, unroll=2)

        for v in range(VPC):
            def ent_body(b, accs):
                a0, a1 = accs
                c0 = cv[pl.ds(2 * b * CHUNK + v * LANES, LANES)]
                c1 = cv[pl.ds((2 * b + 1) * CHUNK + v * LANES, LANES)]
                a0 = a0 + plsc.load_gather(tv, [c0.astype(jnp.int32)])
                a1 = a1 + plsc.load_gather(tv, [c1.astype(jnp.int32)])
                return a0, a1

            acc0, acc1 = lax.fori_loop(0, BINS // 2, ent_body,
                                       (zero16, zero16), unroll=2)
            ev[pl.ds(v * LANES, LANES)] = (
                jnp.full((LANES,), _LOG_BATCH, jnp.float32)
                - (acc0 + acc1) * np.float32(1.0 / BATCH))

        pltpu.sync_copy(ev, out_hbm.at[pl.ds(base, CHUNK)])

    cp0.wait()
    process_chunk(xv0, pl.multiple_of(wid * CHUNK, CHUNK))

    @pl.when(two)
    def _():
        pltpu.make_async_copy(x_hbm.at[wid + NW], xv1, sem1).wait()
        process_chunk(xv1, pl.multiple_of((wid + NW) * CHUNK, CHUNK))


def _sc_hist(x, params, tab, interpret=False):
    mesh = plsc.VectorSubcoreMesh(core_axis_name="c", subcore_axis_name="s")
    return pl.kernel(
        _sc_hist_body,
        out_type=jax.ShapeDtypeStruct((M,), jnp.float32),
        mesh=mesh,
        scratch_types=[
            pltpu.VMEM((BATCH, CHUNK), jnp.float32),
            pltpu.VMEM((BATCH, CHUNK), jnp.float32),
            pltpu.VMEM((2, LANES), jnp.float32),
            pltpu.VMEM((80,), jnp.float32),
            pltpu.VMEM((BINS * CHUNK,), jnp.float32),
            pltpu.VMEM((CHUNK,), jnp.float32),
            pltpu.SemaphoreType.DMA,
            pltpu.SemaphoreType.DMA,
        ],
        compiler_params=pltpu.CompilerParams(needs_layout_passes=False),
        interpret=interpret,
    )(x, params, tab)


def _kernel_impl(tab_tokens, img_tokens, interpret=False):
    x, params = _means_params(tab_tokens, img_tokens, interpret=interpret)
    tab = jnp.asarray(_KLOGK)
    return _sc_hist(x, params, tab, interpret=interpret)


def kernel(tab_tokens, img_tokens):
    return _kernel_impl(tab_tokens, img_tokens)


# speedup vs baseline: 1.1314x; 1.0267x over previous
import functools

import numpy as np
import jax
import jax.numpy as jnp
from jax import lax
from jax.experimental import pallas as pl
from jax.experimental.pallas import tpu as pltpu
from jax.experimental.pallas import tpu_sc as plsc

BINS = 32
BATCH = 64
T_TAB = 2048
T_IMG = 4096
M = T_TAB + T_IMG
FEAT = 128
BT = 256
G_TAB = T_TAB // BT
G_IMG = T_IMG // BT
G = G_TAB + G_IMG

NC = 2
NS = 16
NW = NC * NS
LANES = 16
CHUNK = 128
NCHUNK = M // CHUNK
VPC = CHUNK // LANES

_KLOGK = np.zeros(80, np.float32)
_KLOGK[1:BATCH + 1] = (np.arange(1, BATCH + 1, dtype=np.float64)
                       * np.log(np.arange(1, BATCH + 1, dtype=np.float64))
                       ).astype(np.float32)
_LOG_BATCH = np.float32(np.log(float(BATCH)))


BT_IMG = 512
G_IMG2 = T_IMG // BT_IMG
G2 = G_TAB + G_IMG2
OCH = BT_IMG // CHUNK


def _mean_body(tab_ref, img_ref, x_ref, par_ref, mm_s):
    i = pl.program_id(0)

    def minmax_update(s):
        pmin = jnp.min(s)
        pmax = jnp.max(s)

        @pl.when(i == 0)
        def _():
            mm_s[0, 0] = pmin
            mm_s[0, 1] = pmax

        @pl.when(i > 0)
        def _():
            mm_s[0, 0] = jnp.minimum(mm_s[0, 0], pmin)
            mm_s[0, 1] = jnp.maximum(mm_s[0, 1], pmax)

    @pl.when(i < G_TAB)
    def _():
        s = jnp.sum(tab_ref[...], axis=2) * np.float32(1.0 / FEAT)
        q = i % 2
        for c in range(BT // CHUNK):
            x_ref[pl.ds(q * (BT // CHUNK) + c, 1)] = (
                s[:, c * CHUNK:(c + 1) * CHUNK])[None]
        minmax_update(s)

    @pl.when(i >= G_TAB)
    def _():
        s = jnp.sum(img_ref[...], axis=2) * np.float32(1.0 / FEAT)
        for c in range(OCH):
            x_ref[c] = s[:, c * CHUNK:(c + 1) * CHUNK]
        minmax_update(s)

    @pl.when(i == G2 - 1)
    def _():
        xmin = mm_s[0, 0]
        scale = np.float32(BINS) / (mm_s[0, 1] - xmin)
        par_ref[...] = jnp.stack([
            jnp.full((LANES,), xmin, jnp.float32),
            jnp.full((LANES,), scale, jnp.float32),
        ])


def _means_params(tab_tokens, img_tokens, interpret=False):
    return pl.pallas_call(
        _mean_body,
        grid=(G2,),
        in_specs=[
            pl.BlockSpec((BATCH, BT, FEAT),
                         lambda i: (0, jnp.minimum(i, G_TAB - 1), 0)),
            pl.BlockSpec((BATCH, BT_IMG, FEAT),
                         lambda i: (0, jnp.clip(i - G_TAB, 0, G_IMG2 - 1), 0)),
        ],
        out_specs=[
            pl.BlockSpec((OCH, BATCH, CHUNK),
                         lambda i: (jnp.where(i < G_TAB, i // 2, i - 4), 0, 0)),
            pl.BlockSpec((2, LANES), lambda i: (0, 0)),
        ],
        out_shape=[
            jax.ShapeDtypeStruct((NCHUNK, BATCH, CHUNK), jnp.float32),
            jax.ShapeDtypeStruct((2, LANES), jnp.float32),
        ],
        scratch_shapes=[
            pltpu.SMEM((1, 2), jnp.float32),
        ],
        interpret=interpret,
    )(tab_tokens, img_tokens)


def _sc_hist_body(x_hbm, params_hbm, tab_hbm, out_hbm,
                  xv0, xv1, pv, tv, cv, ev, sem0, sem1):
    wid = lax.axis_index("s") * NC + lax.axis_index("c")
    two = wid + NW < NCHUNK

    cp0 = pltpu.async_copy(x_hbm.at[wid], xv0, sem0)

    @pl.when(two)
    def _():
        pltpu.async_copy(x_hbm.at[wid + NW], xv1, sem1)

    pltpu.sync_copy(params_hbm, pv)
    pltpu.sync_copy(tab_hbm, tv)

    noff = pv[0, :] * pv[1, :]
    scl = pv[1, :]
    zero16 = jnp.zeros((LANES,), jnp.float32)
    one16 = jnp.full((LANES,), 1.0, jnp.float32)
    lane = lax.iota(jnp.int32, LANES)
    cap = jnp.full((LANES,), BINS - 1, jnp.int32)

    def process_chunk(xv, base):
        def zero_body(j, _):
            cv[pl.ds(pl.multiple_of(j * LANES, LANES), LANES)] = zero16
            return 0

        lax.fori_loop(0, BINS * CHUNK // LANES, zero_body, 0, unroll=8)

        def hist_row(r, _):
            for v in range(VPC):
                x = xv[r, pl.ds(v * LANES, LANES)]
                t = x * scl - noff
                b = jnp.minimum(t.astype(jnp.int32), cap)
                flat = b * CHUNK + (lane + v * LANES)
                plsc.addupdate_scatter(cv, [flat], one16)
            return 0

        lax.fori_loop(0, BATCH, hist_row, 0, unroll=2)

        for v in range(VPC):
            def ent_body(b, accs):
                a0, a1 = accs
                c0 = cv[pl.ds(2 * b * CHUNK + v * LANES, LANES)]
                c1 = cv[pl.ds((2 * b + 1) * CHUNK + v * LANES, LANES)]
                a0 = a0 + plsc.load_gather(tv, [c0.astype(jnp.int32)])
                a1 = a1 + plsc.load_gather(tv, [c1.astype(jnp.int32)])
                return a0, a1

            acc0, acc1 = lax.fori_loop(0, BINS // 2, ent_body,
                                       (zero16, zero16), unroll=2)
            ev[pl.ds(v * LANES, LANES)] = (
                jnp.full((LANES,), _LOG_BATCH, jnp.float32)
                - (acc0 + acc1) * np.float32(1.0 / BATCH))

        pltpu.sync_copy(ev, out_hbm.at[pl.ds(base, CHUNK)])

    cp0.wait()
    process_chunk(xv0, pl.multiple_of(wid * CHUNK, CHUNK))

    @pl.when(two)
    def _():
        pltpu.make_async_copy(x_hbm.at[wid + NW], xv1, sem1).wait()
        process_chunk(xv1, pl.multiple_of((wid + NW) * CHUNK, CHUNK))


def _sc_hist(x, params, tab, interpret=False):
    mesh = plsc.VectorSubcoreMesh(core_axis_name="c", subcore_axis_name="s")
    return pl.kernel(
        _sc_hist_body,
        out_type=jax.ShapeDtypeStruct((M,), jnp.float32),
        mesh=mesh,
        scratch_types=[
            pltpu.VMEM((BATCH, CHUNK), jnp.float32),
            pltpu.VMEM((BATCH, CHUNK), jnp.float32),
            pltpu.VMEM((2, LANES), jnp.float32),
            pltpu.VMEM((80,), jnp.float32),
            pltpu.VMEM((BINS * CHUNK,), jnp.float32),
            pltpu.VMEM((CHUNK,), jnp.float32),
            pltpu.SemaphoreType.DMA,
            pltpu.SemaphoreType.DMA,
        ],
        compiler_params=pltpu.CompilerParams(needs_layout_passes=False),
        interpret=interpret,
    )(x, params, tab)


def _kernel_impl(tab_tokens, img_tokens, interpret=False):
    x, params = _means_params(tab_tokens, img_tokens, interpret=interpret)
    tab = jnp.asarray(_KLOGK)
    return _sc_hist(x, params, tab, interpret=interpret)


def kernel(tab_tokens, img_tokens):
    return _kernel_impl(tab_tokens, img_tokens)
